# Initial kernel scaffold; baseline (speedup 1.0000x reference)
import jax
import jax.numpy as jnp
from jax.experimental import pallas as pl


def kernel(x, edge_index, prompt, W1, b1, W2, b2, Wp, bp, Wm1, bm1, g1, be1, Wm2, bm2, g2, be2, Wout, bout):
    raise NotImplementedError("write your pallas kernel here")


# speedup vs baseline: 4.9302x; 4.9302x over previous
import functools

import jax
import jax.numpy as jnp
from jax import lax
from jax.experimental import pallas as pl
from jax.experimental.pallas import tpu as pltpu
from jax.experimental.pallas import tpu_sc as plsc

_NC = 2
_NS = 16
_NW = _NC * _NS
_LANES = 16
_BN_EPS = 1e-5


@functools.lru_cache(maxsize=None)
def _make_segsum(N, D, E, CH):
    EPW = E // _NW
    NCH = EPW // CH
    BASE = (N // _NS) & ~7
    EXTRA = N - _NS * BASE
    ZR = 104
    assert EPW * _NW == E and NCH * CH == EPW
    assert CH % 8 == 0 and CH <= 128 and EPW % 8 == 0
    assert BASE % ZR == 0 and EXTRA % 8 == 0 and EXTRA <= ZR
    mesh = plsc.VectorSubcoreMesh(core_axis_name="c", subcore_axis_name="s")

    @functools.partial(
        pl.kernel,
        out_type=jax.ShapeDtypeStruct((_NC * N, D), jnp.float32),
        mesh=mesh,
        scratch_types=[
            pltpu.VMEM((EPW,), jnp.int32),
            pltpu.VMEM((CH,), jnp.int32),
            pltpu.VMEM((CH, D), jnp.float32),
            pltpu.VMEM((ZR, D), jnp.float32),
            pltpu.VMEM_SHARED((N, D), jnp.float32),
            pltpu.SemaphoreType.DMA,
        ],
    )
    def segsum(h_hbm, src_hbm, dst_hbm, out_hbm, src_v, dst_v, rows_v, zbuf,
               acc, sem):
        c = lax.axis_index("c")
        s = lax.axis_index("s")
        wid = c * _NS + s

        zeros = jnp.zeros((_LANES,), jnp.float32)

        def zrow(i, carry):
            for q in range(D // _LANES):
                zbuf[i, pl.ds(q * _LANES, _LANES)] = zeros
            return carry

        lax.fori_loop(0, ZR, zrow, 0)

        def zacc(t, carry):
            pltpu.sync_copy(zbuf, acc.at[pl.ds(s * BASE + t * ZR, ZR)])
            return carry

        lax.fori_loop(0, BASE // ZR, zacc, 0)

        @pl.when(s == _NS - 1)
        def _():
            pltpu.sync_copy(zbuf.at[pl.ds(0, EXTRA)],
                            acc.at[pl.ds(_NS * BASE, EXTRA)])

        pltpu.sync_copy(src_hbm.at[pl.ds(wid * EPW, EPW)], src_v)
        plsc.subcore_barrier()

        def body(j, carry):
            pltpu.sync_copy(dst_hbm.at[pl.ds(wid * EPW + j * CH, CH)], dst_v)
            pltpu.async_copy(h_hbm.at[src_v.at[pl.ds(j * CH, CH)]], rows_v,
                             sem).wait()
            pltpu.sync_copy(rows_v, acc.at[dst_v], add=True)
            return carry

        lax.fori_loop(0, NCH, body, 0)
        plsc.subcore_barrier()

        pltpu.sync_copy(acc.at[pl.ds(s * BASE, BASE)],
                        out_hbm.at[pl.ds(c * N + s * BASE, BASE)])

        @pl.when(s == _NS - 1)
        def _():
            pltpu.sync_copy(acc.at[pl.ds(_NS * BASE, EXTRA)],
                            out_hbm.at[pl.ds(c * N + _NS * BASE, EXTRA)])

    return segsum


def _gin_dense(x, parts, W, b, BM):
    N, D = x.shape
    NB = N // BM

    def body(x_ref, pa_ref, pb_ref, w_ref, b_ref, o_ref):
        h = x_ref[...] + pa_ref[...] + pb_ref[...]
        z = jnp.dot(h, w_ref[...], preferred_element_type=jnp.float32)
        o_ref[...] = jnp.maximum(z + b_ref[...], 0.0)

    return pl.pallas_call(
        body,
        grid=(NB,),
        in_specs=[
            pl.BlockSpec((BM, D), lambda i: (i, 0)),
            pl.BlockSpec((BM, D), lambda i: (i, 0)),
            pl.BlockSpec((BM, D), lambda i: (i + NB, 0)),
            pl.BlockSpec((D, D), lambda i: (0, 0)),
            pl.BlockSpec((1, D), lambda i: (0, 0)),
        ],
        out_specs=pl.BlockSpec((BM, D), lambda i: (i, 0)),
        out_shape=jax.ShapeDtypeStruct((N, D), jnp.float32),
    )(x, parts, parts, W, b.reshape(1, D))


def _gin2_prompt_mlp1(h1, parts, W2, b2, prompt, Wp, bp, Wm1h, Wm1p, bm1, BM):
    N, D = h1.shape
    P = prompt.shape[1]
    H1 = Wm1h.shape[1]
    NB = N // BM

    def body(h_ref, pa_ref, pb_ref, w2_ref, b2_ref, pr_ref, wp_ref, bp_ref,
             wh_ref, wpp_ref, bm_ref, z_ref, st_ref):
        i = pl.program_id(0)
        h = h_ref[...] + pa_ref[...] + pb_ref[...]
        h2 = jnp.dot(h, w2_ref[...], preferred_element_type=jnp.float32)
        h2 = jnp.maximum(h2 + b2_ref[...], 0.0)
        pr = jnp.dot(pr_ref[...], wp_ref[...],
                     preferred_element_type=jnp.float32) + bp_ref[...]
        z = (jnp.dot(h2, wh_ref[...], preferred_element_type=jnp.float32)
             + jnp.dot(pr, wpp_ref[...], preferred_element_type=jnp.float32)
             + bm_ref[...])
        z_ref[...] = z
        blk = jnp.concatenate(
            [jnp.sum(z, axis=0)[None, :], jnp.sum(z * z, axis=0)[None, :]], 0)

        @pl.when(i == 0)
        def _():
            st_ref[...] = jnp.zeros_like(st_ref)

        st_ref[...] += blk

    return pl.pallas_call(
        body,
        grid=(NB,),
        in_specs=[
            pl.BlockSpec((BM, D), lambda i: (i, 0)),
            pl.BlockSpec((BM, D), lambda i: (i, 0)),
            pl.BlockSpec((BM, D), lambda i: (i + NB, 0)),
            pl.BlockSpec((D, D), lambda i: (0, 0)),
            pl.BlockSpec((1, D), lambda i: (0, 0)),
            pl.BlockSpec((BM, P), lambda i: (i, 0)),
            pl.BlockSpec((P, P), lambda i: (0, 0)),
            pl.BlockSpec((1, P), lambda i: (0, 0)),
            pl.BlockSpec((D, H1), lambda i: (0, 0)),
            pl.BlockSpec((P, H1), lambda i: (0, 0)),
            pl.BlockSpec((1, H1), lambda i: (0, 0)),
        ],
        out_specs=[
            pl.BlockSpec((BM, H1), lambda i: (i, 0)),
            pl.BlockSpec((2, H1), lambda i: (0, 0)),
        ],
        out_shape=[
            jax.ShapeDtypeStruct((N, H1), jnp.float32),
            jax.ShapeDtypeStruct((2, H1), jnp.float32),
        ],
    )(h1, parts, parts, W2, b2.reshape(1, D), prompt, Wp, bp.reshape(1, P),
      Wm1h, Wm1p, bm1.reshape(1, H1))


def _bn_relu_matmul(z, st, g, be, W, b, BM, with_stats):
    N, H = z.shape
    HO = W.shape[1]
    NB = N // BM

    def body(z_ref, st_ref, g_ref, be_ref, w_ref, b_ref, *out_refs):
        i = pl.program_id(0)
        m = st_ref[0, :] / N
        v = st_ref[1, :] / N - m * m
        r = lax.rsqrt(v + _BN_EPS)
        a = (z_ref[...] - m[None, :]) * (r * g_ref[0, :])[None, :] + be_ref[...]
        a = jnp.maximum(a, 0.0)
        o = jnp.dot(a, w_ref[...], preferred_element_type=jnp.float32)
        o = o + b_ref[...]
        out_refs[0][...] = o
        if with_stats:
            blk = jnp.concatenate(
                [jnp.sum(o, axis=0)[None, :], jnp.sum(o * o, axis=0)[None, :]],
                0)

            @pl.when(i == 0)
            def _():
                out_refs[1][...] = jnp.zeros_like(out_refs[1])

            out_refs[1][...] += blk

    out_specs = [pl.BlockSpec((BM, HO), lambda i: (i, 0))]
    out_shape = [jax.ShapeDtypeStruct((N, HO), jnp.float32)]
    if with_stats:
        out_specs.append(pl.BlockSpec((2, HO), lambda i: (0, 0)))
        out_shape.append(jax.ShapeDtypeStruct((2, HO), jnp.float32))

    res = pl.pallas_call(
        body,
        grid=(NB,),
        in_specs=[
            pl.BlockSpec((BM, H), lambda i: (i, 0)),
            pl.BlockSpec((2, H), lambda i: (0, 0)),
            pl.BlockSpec((1, H), lambda i: (0, 0)),
            pl.BlockSpec((1, H), lambda i: (0, 0)),
            pl.BlockSpec((H, HO), lambda i: (0, 0)),
            pl.BlockSpec((1, HO), lambda i: (0, 0)),
        ],
        out_specs=out_specs,
        out_shape=out_shape,
    )(z, st, g.reshape(1, H), be.reshape(1, H), W, b.reshape(1, HO))
    return res if with_stats else res[0]


def kernel(x, edge_index, prompt, W1, b1, W2, b2, Wp, bp,
           Wm1, bm1, g1, be1, Wm2, bm2, g2, be2, Wout, bout):
    N, D = x.shape
    E = edge_index.shape[1]
    CH = 80
    BM = 2000

    src = edge_index[0].astype(jnp.int32)
    dst = edge_index[1].astype(jnp.int32)

    segsum = _make_segsum(N, D, E, CH)

    parts1 = segsum(x, src, dst)
    h1 = _gin_dense(x, parts1, W1, b1, BM)
    parts2 = segsum(h1, src, dst)
    z1, st1 = _gin2_prompt_mlp1(h1, parts2, W2, b2, prompt, Wp, bp,
                                Wm1[:D], Wm1[D:], bm1, BM)
    z2, st2 = _bn_relu_matmul(z1, st1, g1, be1, Wm2, bm2, BM, True)
    out = _bn_relu_matmul(z2, st2, g2, be2, Wout, bout, BM, False)
    return out
